# baseline (speedup 1.0000x reference)
import jax
import jax.numpy as jnp
from jax import lax
from jax.experimental import pallas as pl
from jax.experimental.pallas import tpu as pltpu
from jax.experimental.pallas import tpu_sc as plsc

_N_ROWS = 65536
_N_COLS = 512
_ONES_ROWS = 512
_BLOCK_ROWS = 2048


def _fill_kernel(o_ref):
    i = pl.program_id(0)
    row = jax.lax.broadcasted_iota(jnp.int32, o_ref.shape, 0) + i * _BLOCK_ROWS
    o_ref[...] = (row < _ONES_ROWS).astype(jnp.float32)


def _tc_kernel(x):
    return pl.pallas_call(
        _fill_kernel,
        grid=(_N_ROWS // _BLOCK_ROWS,),
        out_specs=pl.BlockSpec((_BLOCK_ROWS, _N_COLS), lambda i: (i, 0)),
        out_shape=jax.ShapeDtypeStruct((_N_ROWS, _N_COLS), x.dtype),
        compiler_params=pltpu.CompilerParams(
            dimension_semantics=("parallel",),
        ),
    )()


_NC = 2
_NS = 16
_NW = _NC * _NS
_ROWS_PER_W = _N_ROWS // _NW
_TPL_ROWS = 128
_COPIES_PER_W = _ROWS_PER_W // _TPL_ROWS
_ONES_COPIES = _ONES_ROWS // _TPL_ROWS


def _sc_body(o_hbm, tpl, sem):
    c = lax.axis_index("c")
    s = lax.axis_index("s")
    wid = s * _NC + c
    base = wid * _ROWS_PER_W

    def fill(val):
        def row_body(r, carry):
            for cc in range(_N_COLS // 16):
                tpl[r, pl.ds(cc * 16, 16)] = jnp.full((16,), val, jnp.float32)
            return carry
        lax.fori_loop(0, _TPL_ROWS, row_body, 0)

    @pl.when(wid == 0)
    def _():
        fill(1.0)
        cps = [
            pltpu.make_async_copy(
                tpl, o_hbm.at[pl.ds(k * _TPL_ROWS, _TPL_ROWS), :], sem
            )
            for k in range(_ONES_COPIES)
        ]
        for cp in cps:
            cp.start()
        for cp in cps:
            cp.wait()
        fill(0.0)
        cps = [
            pltpu.make_async_copy(
                tpl,
                o_hbm.at[pl.ds(_ONES_ROWS + k * _TPL_ROWS, _TPL_ROWS), :],
                sem,
            )
            for k in range(_COPIES_PER_W - _ONES_COPIES)
        ]
        for cp in cps:
            cp.start()
        for cp in cps:
            cp.wait()

    @pl.when(wid != 0)
    def _():
        fill(0.0)
        cps = [
            pltpu.make_async_copy(
                tpl,
                o_hbm.at[pl.ds(base + k * _TPL_ROWS, _TPL_ROWS), :],
                sem,
            )
            for k in range(_COPIES_PER_W)
        ]
        for cp in cps:
            cp.start()
        for cp in cps:
            cp.wait()


def _sc_kernel(x):
    mesh = plsc.VectorSubcoreMesh(core_axis_name="c", subcore_axis_name="s")
    run = pl.kernel(
        _sc_body,
        out_type=jax.ShapeDtypeStruct((_N_ROWS, _N_COLS), x.dtype),
        mesh=mesh,
        scratch_types=[
            pltpu.VMEM((_TPL_ROWS, _N_COLS), jnp.float32),
            pltpu.SemaphoreType.DMA,
        ],
    )
    return run()


def kernel(x):
    return _sc_kernel(x)


# speedup vs baseline: 1.0342x; 1.0342x over previous
import jax
import jax.numpy as jnp
from jax import lax
from jax.experimental import pallas as pl
from jax.experimental.pallas import tpu as pltpu
from jax.experimental.pallas import tpu_sc as plsc

_N_ROWS = 65536
_N_COLS = 512
_ONES_ROWS = 512
_BLOCK_ROWS = 2048


def _fill_kernel(o_ref):
    i = pl.program_id(0)
    row = jax.lax.broadcasted_iota(jnp.int32, o_ref.shape, 0) + i * _BLOCK_ROWS
    o_ref[...] = (row < _ONES_ROWS).astype(jnp.float32)


def _tc_kernel(x):
    return pl.pallas_call(
        _fill_kernel,
        grid=(_N_ROWS // _BLOCK_ROWS,),
        out_specs=pl.BlockSpec((_BLOCK_ROWS, _N_COLS), lambda i: (i, 0)),
        out_shape=jax.ShapeDtypeStruct((_N_ROWS, _N_COLS), x.dtype),
        compiler_params=pltpu.CompilerParams(
            dimension_semantics=("parallel",),
        ),
    )()


_NC = 2
_NS = 16
_NW = _NC * _NS
_ROWS_PER_W = _N_ROWS // _NW
_TPL_ROWS = 128
_COPIES_PER_W = _ROWS_PER_W // _TPL_ROWS
_ONES_COPIES = _ONES_ROWS // _TPL_ROWS


def _sc_body(o_hbm, tpl, sem):
    c = lax.axis_index("c")
    s = lax.axis_index("s")
    wid = s * _NC + c
    base = wid * _ROWS_PER_W

    def fill(val):
        def row_body(r, carry):
            for cc in range(_N_COLS // 16):
                tpl[r, pl.ds(cc * 16, 16)] = jnp.full((16,), val, jnp.float32)
            return carry
        lax.fori_loop(0, _TPL_ROWS, row_body, 0)

    @pl.when(wid == 0)
    def _():
        fill(1.0)
        cps = [
            pltpu.make_async_copy(
                tpl, o_hbm.at[pl.ds(k * _TPL_ROWS, _TPL_ROWS), :], sem
            )
            for k in range(_ONES_COPIES)
        ]
        for cp in cps:
            cp.start()
        for cp in cps:
            cp.wait()
        fill(0.0)
        cps = [
            pltpu.make_async_copy(
                tpl,
                o_hbm.at[pl.ds(_ONES_ROWS + k * _TPL_ROWS, _TPL_ROWS), :],
                sem,
            )
            for k in range(_COPIES_PER_W - _ONES_COPIES)
        ]
        for cp in cps:
            cp.start()
        for cp in cps:
            cp.wait()

    @pl.when(wid != 0)
    def _():
        fill(0.0)
        cps = [
            pltpu.make_async_copy(
                tpl,
                o_hbm.at[pl.ds(base + k * _TPL_ROWS, _TPL_ROWS), :],
                sem,
            )
            for k in range(_COPIES_PER_W)
        ]
        for cp in cps:
            cp.start()
        for cp in cps:
            cp.wait()


def _sc_kernel(x):
    mesh = plsc.VectorSubcoreMesh(core_axis_name="c", subcore_axis_name="s")
    run = pl.kernel(
        _sc_body,
        out_type=jax.ShapeDtypeStruct((_N_ROWS, _N_COLS), x.dtype),
        mesh=mesh,
        scratch_types=[
            pltpu.VMEM((_TPL_ROWS, _N_COLS), jnp.float32),
            pltpu.SemaphoreType.DMA,
        ],
    )
    return run()


_SC_ROWS_PER_W = _ONES_ROWS // _NW


def _sc_ones_body(o_hbm, tpl, sem):
    c = lax.axis_index("c")
    s = lax.axis_index("s")
    wid = s * _NC + c
    for r in range(_SC_ROWS_PER_W):
        for cc in range(_N_COLS // 16):
            tpl[r, pl.ds(cc * 16, 16)] = jnp.full((16,), 1.0, jnp.float32)
    cp = pltpu.make_async_copy(
        tpl, o_hbm.at[pl.ds(wid * _SC_ROWS_PER_W, _SC_ROWS_PER_W), :], sem
    )
    cp.start()
    cp.wait()


def _sc_ones(dtype):
    mesh = plsc.VectorSubcoreMesh(core_axis_name="c", subcore_axis_name="s")
    run = pl.kernel(
        _sc_ones_body,
        out_type=jax.ShapeDtypeStruct((_ONES_ROWS, _N_COLS), dtype),
        mesh=mesh,
        scratch_types=[
            pltpu.VMEM((_SC_ROWS_PER_W, _N_COLS), jnp.float32),
            pltpu.SemaphoreType.DMA,
        ],
    )
    return run()


def _tc_zero_init_body(ones_ref, o_ref):
    i = pl.program_id(0)
    o_ref[...] = jnp.zeros(o_ref.shape, jnp.float32)

    @pl.when(i == 0)
    def _():
        o_ref[0:_ONES_ROWS, :] = ones_ref[...]


def _hybrid_kernel(x):
    ones_block = _sc_ones(x.dtype)
    return pl.pallas_call(
        _tc_zero_init_body,
        grid=(_N_ROWS // _BLOCK_ROWS,),
        in_specs=[pl.BlockSpec((_ONES_ROWS, _N_COLS), lambda i: (0, 0))],
        out_specs=pl.BlockSpec((_BLOCK_ROWS, _N_COLS), lambda i: (i, 0)),
        out_shape=jax.ShapeDtypeStruct((_N_ROWS, _N_COLS), x.dtype),
    )(ones_block)


def kernel(x):
    return _hybrid_kernel(x)


# speedup vs baseline: 1.0997x; 1.0633x over previous
import jax
import jax.numpy as jnp
from jax import lax
from jax.experimental import pallas as pl
from jax.experimental.pallas import tpu as pltpu
from jax.experimental.pallas import tpu_sc as plsc

_N_ROWS = 65536
_N_COLS = 512
_ONES_ROWS = 512
_BLOCK_ROWS = 2048


def _fill_kernel(o_ref):
    i = pl.program_id(0)
    row = jax.lax.broadcasted_iota(jnp.int32, o_ref.shape, 0) + i * _BLOCK_ROWS
    o_ref[...] = (row < _ONES_ROWS).astype(jnp.float32)


def _tc_kernel(x):
    return pl.pallas_call(
        _fill_kernel,
        grid=(_N_ROWS // _BLOCK_ROWS,),
        out_specs=pl.BlockSpec((_BLOCK_ROWS, _N_COLS), lambda i: (i, 0)),
        out_shape=jax.ShapeDtypeStruct((_N_ROWS, _N_COLS), x.dtype),
        compiler_params=pltpu.CompilerParams(
            dimension_semantics=("parallel",),
        ),
    )()


_NC = 2
_NS = 16
_NW = _NC * _NS
_ROWS_PER_W = _N_ROWS // _NW
_TPL_ROWS = 128
_COPIES_PER_W = _ROWS_PER_W // _TPL_ROWS
_ONES_COPIES = _ONES_ROWS // _TPL_ROWS


def _sc_body(o_hbm, tpl, sem):
    c = lax.axis_index("c")
    s = lax.axis_index("s")
    wid = s * _NC + c
    base = wid * _ROWS_PER_W

    def fill(val):
        def row_body(r, carry):
            for cc in range(_N_COLS // 16):
                tpl[r, pl.ds(cc * 16, 16)] = jnp.full((16,), val, jnp.float32)
            return carry
        lax.fori_loop(0, _TPL_ROWS, row_body, 0)

    @pl.when(wid == 0)
    def _():
        fill(1.0)
        cps = [
            pltpu.make_async_copy(
                tpl, o_hbm.at[pl.ds(k * _TPL_ROWS, _TPL_ROWS), :], sem
            )
            for k in range(_ONES_COPIES)
        ]
        for cp in cps:
            cp.start()
        for cp in cps:
            cp.wait()
        fill(0.0)
        cps = [
            pltpu.make_async_copy(
                tpl,
                o_hbm.at[pl.ds(_ONES_ROWS + k * _TPL_ROWS, _TPL_ROWS), :],
                sem,
            )
            for k in range(_COPIES_PER_W - _ONES_COPIES)
        ]
        for cp in cps:
            cp.start()
        for cp in cps:
            cp.wait()

    @pl.when(wid != 0)
    def _():
        fill(0.0)
        cps = [
            pltpu.make_async_copy(
                tpl,
                o_hbm.at[pl.ds(base + k * _TPL_ROWS, _TPL_ROWS), :],
                sem,
            )
            for k in range(_COPIES_PER_W)
        ]
        for cp in cps:
            cp.start()
        for cp in cps:
            cp.wait()


def _sc_kernel(x):
    mesh = plsc.VectorSubcoreMesh(core_axis_name="c", subcore_axis_name="s")
    run = pl.kernel(
        _sc_body,
        out_type=jax.ShapeDtypeStruct((_N_ROWS, _N_COLS), x.dtype),
        mesh=mesh,
        scratch_types=[
            pltpu.VMEM((_TPL_ROWS, _N_COLS), jnp.float32),
            pltpu.SemaphoreType.DMA,
        ],
    )
    return run()


_SC_ROWS_PER_W = _ONES_ROWS // _NW


def _sc_ones_body(o_hbm, tpl, sem):
    c = lax.axis_index("c")
    s = lax.axis_index("s")
    wid = s * _NC + c
    for r in range(_SC_ROWS_PER_W):
        for cc in range(_N_COLS // 16):
            tpl[r, pl.ds(cc * 16, 16)] = jnp.full((16,), 1.0, jnp.float32)
    cp = pltpu.make_async_copy(
        tpl, o_hbm.at[pl.ds(wid * _SC_ROWS_PER_W, _SC_ROWS_PER_W), :], sem
    )
    cp.start()
    cp.wait()


def _sc_ones(dtype):
    mesh = plsc.VectorSubcoreMesh(core_axis_name="c", subcore_axis_name="s")
    run = pl.kernel(
        _sc_ones_body,
        out_type=jax.ShapeDtypeStruct((_ONES_ROWS, _N_COLS), dtype),
        mesh=mesh,
        scratch_types=[
            pltpu.VMEM((_SC_ROWS_PER_W, _N_COLS), jnp.float32),
            pltpu.SemaphoreType.DMA,
        ],
    )
    return run()


def _tc_zero_init_body(ones_ref, o_ref):
    i = pl.program_id(0)
    o_ref[...] = jnp.zeros(o_ref.shape, jnp.float32)

    @pl.when(i == 0)
    def _():
        o_ref[0:_ONES_ROWS, :] = ones_ref[...]


def _hybrid_kernel(x):
    ones_block = _sc_ones(x.dtype)
    return pl.pallas_call(
        _tc_zero_init_body,
        grid=(_N_ROWS // _BLOCK_ROWS,),
        in_specs=[pl.BlockSpec((_ONES_ROWS, _N_COLS), lambda i: (0, 0))],
        out_specs=pl.BlockSpec((_BLOCK_ROWS, _N_COLS), lambda i: (i, 0)),
        out_shape=jax.ShapeDtypeStruct((_N_ROWS, _N_COLS), x.dtype),
    )(ones_block)


def _tc_zeros_body(o_ref):
    o_ref[...] = jnp.zeros(o_ref.shape, jnp.float32)


def _tc_zeros(dtype):
    return pl.pallas_call(
        _tc_zeros_body,
        grid=(_N_ROWS // _BLOCK_ROWS,),
        out_specs=pl.BlockSpec((_BLOCK_ROWS, _N_COLS), lambda i: (i, 0)),
        out_shape=jax.ShapeDtypeStruct((_N_ROWS, _N_COLS), dtype),
    )()


def _merge_body(z_ref, ones_ref, o_ref):
    o_ref[...] = z_ref[...] + ones_ref[...]


def _merge(zeros, ones_block):
    return pl.pallas_call(
        _merge_body,
        grid=(1,),
        in_specs=[
            pl.BlockSpec((_ONES_ROWS, _N_COLS), lambda i: (0, 0)),
            pl.BlockSpec((_ONES_ROWS, _N_COLS), lambda i: (0, 0)),
        ],
        out_specs=pl.BlockSpec((_ONES_ROWS, _N_COLS), lambda i: (0, 0)),
        out_shape=jax.ShapeDtypeStruct((_N_ROWS, _N_COLS), zeros.dtype),
        input_output_aliases={0: 0},
    )(zeros, ones_block)


def _hybrid2_kernel(x):
    ones_block = _sc_ones(x.dtype)
    zeros = _tc_zeros(x.dtype)
    return _merge(zeros, ones_block)


def kernel(x):
    return _hybrid2_kernel(x)


# speedup vs baseline: 1.1052x; 1.0051x over previous
import jax
import jax.numpy as jnp
from jax import lax
from jax.experimental import pallas as pl
from jax.experimental.pallas import tpu as pltpu
from jax.experimental.pallas import tpu_sc as plsc

_N_ROWS = 65536
_N_COLS = 512
_ONES_ROWS = 512
_BLOCK_ROWS = 2048


def _fill_kernel(o_ref):
    i = pl.program_id(0)
    row = jax.lax.broadcasted_iota(jnp.int32, o_ref.shape, 0) + i * _BLOCK_ROWS
    o_ref[...] = (row < _ONES_ROWS).astype(jnp.float32)


def _tc_kernel(x):
    return pl.pallas_call(
        _fill_kernel,
        grid=(_N_ROWS // _BLOCK_ROWS,),
        out_specs=pl.BlockSpec((_BLOCK_ROWS, _N_COLS), lambda i: (i, 0)),
        out_shape=jax.ShapeDtypeStruct((_N_ROWS, _N_COLS), x.dtype),
        compiler_params=pltpu.CompilerParams(
            dimension_semantics=("parallel",),
        ),
    )()


_NC = 2
_NS = 16
_NW = _NC * _NS
_ROWS_PER_W = _N_ROWS // _NW
_TPL_ROWS = 128
_COPIES_PER_W = _ROWS_PER_W // _TPL_ROWS
_ONES_COPIES = _ONES_ROWS // _TPL_ROWS


def _sc_body(o_hbm, tpl, sem):
    c = lax.axis_index("c")
    s = lax.axis_index("s")
    wid = s * _NC + c
    base = wid * _ROWS_PER_W

    def fill(val):
        def row_body(r, carry):
            for cc in range(_N_COLS // 16):
                tpl[r, pl.ds(cc * 16, 16)] = jnp.full((16,), val, jnp.float32)
            return carry
        lax.fori_loop(0, _TPL_ROWS, row_body, 0)

    @pl.when(wid == 0)
    def _():
        fill(1.0)
        cps = [
            pltpu.make_async_copy(
                tpl, o_hbm.at[pl.ds(k * _TPL_ROWS, _TPL_ROWS), :], sem
            )
            for k in range(_ONES_COPIES)
        ]
        for cp in cps:
            cp.start()
        for cp in cps:
            cp.wait()
        fill(0.0)
        cps = [
            pltpu.make_async_copy(
                tpl,
                o_hbm.at[pl.ds(_ONES_ROWS + k * _TPL_ROWS, _TPL_ROWS), :],
                sem,
            )
            for k in range(_COPIES_PER_W - _ONES_COPIES)
        ]
        for cp in cps:
            cp.start()
        for cp in cps:
            cp.wait()

    @pl.when(wid != 0)
    def _():
        fill(0.0)
        cps = [
            pltpu.make_async_copy(
                tpl,
                o_hbm.at[pl.ds(base + k * _TPL_ROWS, _TPL_ROWS), :],
                sem,
            )
            for k in range(_COPIES_PER_W)
        ]
        for cp in cps:
            cp.start()
        for cp in cps:
            cp.wait()


def _sc_kernel(x):
    mesh = plsc.VectorSubcoreMesh(core_axis_name="c", subcore_axis_name="s")
    run = pl.kernel(
        _sc_body,
        out_type=jax.ShapeDtypeStruct((_N_ROWS, _N_COLS), x.dtype),
        mesh=mesh,
        scratch_types=[
            pltpu.VMEM((_TPL_ROWS, _N_COLS), jnp.float32),
            pltpu.SemaphoreType.DMA,
        ],
    )
    return run()


_SC_ROWS_PER_W = _ONES_ROWS // _NW


def _sc_ones_body(o_hbm, tpl, sem):
    c = lax.axis_index("c")
    s = lax.axis_index("s")
    wid = s * _NC + c
    for r in range(_SC_ROWS_PER_W):
        for cc in range(_N_COLS // 16):
            tpl[r, pl.ds(cc * 16, 16)] = jnp.full((16,), 1.0, jnp.float32)
    cp = pltpu.make_async_copy(
        tpl, o_hbm.at[pl.ds(wid * _SC_ROWS_PER_W, _SC_ROWS_PER_W), :], sem
    )
    cp.start()
    cp.wait()


def _sc_ones(dtype):
    mesh = plsc.VectorSubcoreMesh(core_axis_name="c", subcore_axis_name="s")
    run = pl.kernel(
        _sc_ones_body,
        out_type=jax.ShapeDtypeStruct((_ONES_ROWS, _N_COLS), dtype),
        mesh=mesh,
        scratch_types=[
            pltpu.VMEM((_SC_ROWS_PER_W, _N_COLS), jnp.float32),
            pltpu.SemaphoreType.DMA,
        ],
    )
    return run()


def _tc_zero_init_body(ones_ref, o_ref):
    i = pl.program_id(0)
    o_ref[...] = jnp.zeros(o_ref.shape, jnp.float32)

    @pl.when(i == 0)
    def _():
        o_ref[0:_ONES_ROWS, :] = ones_ref[...]


def _hybrid_kernel(x):
    ones_block = _sc_ones(x.dtype)
    return pl.pallas_call(
        _tc_zero_init_body,
        grid=(_N_ROWS // _BLOCK_ROWS,),
        in_specs=[pl.BlockSpec((_ONES_ROWS, _N_COLS), lambda i: (0, 0))],
        out_specs=pl.BlockSpec((_BLOCK_ROWS, _N_COLS), lambda i: (i, 0)),
        out_shape=jax.ShapeDtypeStruct((_N_ROWS, _N_COLS), x.dtype),
    )(ones_block)


def _tc_zeros_body(o_ref):
    o_ref[...] = jnp.zeros(o_ref.shape, jnp.float32)


def _tc_zeros(dtype):
    return pl.pallas_call(
        _tc_zeros_body,
        grid=(_N_ROWS // _BLOCK_ROWS,),
        out_specs=pl.BlockSpec((_BLOCK_ROWS, _N_COLS), lambda i: (i, 0)),
        out_shape=jax.ShapeDtypeStruct((_N_ROWS, _N_COLS), dtype),
    )()


def _merge_body(z_ref, ones_ref, o_ref):
    del z_ref
    o_ref[...] = ones_ref[...]


def _merge(zeros, ones_block):
    return pl.pallas_call(
        _merge_body,
        grid=(1,),
        in_specs=[
            pl.BlockSpec(memory_space=pl.ANY),
            pl.BlockSpec((_ONES_ROWS, _N_COLS), lambda i: (0, 0)),
        ],
        out_specs=pl.BlockSpec((_ONES_ROWS, _N_COLS), lambda i: (0, 0)),
        out_shape=jax.ShapeDtypeStruct((_N_ROWS, _N_COLS), zeros.dtype),
        input_output_aliases={0: 0},
    )(zeros, ones_block)


def _hybrid2_kernel(x):
    ones_block = _sc_ones(x.dtype)
    zeros = _tc_zeros(x.dtype)
    return _merge(zeros, ones_block)


def kernel(x):
    return _hybrid2_kernel(x)


# speedup vs baseline: 1.1388x; 1.0304x over previous
import jax
import jax.numpy as jnp
from jax import lax
from jax.experimental import pallas as pl
from jax.experimental.pallas import tpu as pltpu
from jax.experimental.pallas import tpu_sc as plsc

_N_ROWS = 65536
_N_COLS = 512
_ONES_ROWS = 512
_BLOCK_ROWS = 2048


def _fill_kernel(o_ref):
    i = pl.program_id(0)
    row = jax.lax.broadcasted_iota(jnp.int32, o_ref.shape, 0) + i * _BLOCK_ROWS
    o_ref[...] = (row < _ONES_ROWS).astype(jnp.float32)


def _tc_kernel(x):
    return pl.pallas_call(
        _fill_kernel,
        grid=(_N_ROWS // _BLOCK_ROWS,),
        out_specs=pl.BlockSpec((_BLOCK_ROWS, _N_COLS), lambda i: (i, 0)),
        out_shape=jax.ShapeDtypeStruct((_N_ROWS, _N_COLS), x.dtype),
        compiler_params=pltpu.CompilerParams(
            dimension_semantics=("parallel",),
        ),
    )()


_NC = 2
_NS = 16
_NW = _NC * _NS
_ROWS_PER_W = _N_ROWS // _NW
_TPL_ROWS = 128
_COPIES_PER_W = _ROWS_PER_W // _TPL_ROWS
_ONES_COPIES = _ONES_ROWS // _TPL_ROWS


def _sc_body(o_hbm, tpl, sem):
    c = lax.axis_index("c")
    s = lax.axis_index("s")
    wid = s * _NC + c
    base = wid * _ROWS_PER_W

    def fill(val):
        def row_body(r, carry):
            for cc in range(_N_COLS // 16):
                tpl[r, pl.ds(cc * 16, 16)] = jnp.full((16,), val, jnp.float32)
            return carry
        lax.fori_loop(0, _TPL_ROWS, row_body, 0)

    @pl.when(wid == 0)
    def _():
        fill(1.0)
        cps = [
            pltpu.make_async_copy(
                tpl, o_hbm.at[pl.ds(k * _TPL_ROWS, _TPL_ROWS), :], sem
            )
            for k in range(_ONES_COPIES)
        ]
        for cp in cps:
            cp.start()
        for cp in cps:
            cp.wait()
        fill(0.0)
        cps = [
            pltpu.make_async_copy(
                tpl,
                o_hbm.at[pl.ds(_ONES_ROWS + k * _TPL_ROWS, _TPL_ROWS), :],
                sem,
            )
            for k in range(_COPIES_PER_W - _ONES_COPIES)
        ]
        for cp in cps:
            cp.start()
        for cp in cps:
            cp.wait()

    @pl.when(wid != 0)
    def _():
        fill(0.0)
        cps = [
            pltpu.make_async_copy(
                tpl,
                o_hbm.at[pl.ds(base + k * _TPL_ROWS, _TPL_ROWS), :],
                sem,
            )
            for k in range(_COPIES_PER_W)
        ]
        for cp in cps:
            cp.start()
        for cp in cps:
            cp.wait()


def _sc_kernel(x):
    mesh = plsc.VectorSubcoreMesh(core_axis_name="c", subcore_axis_name="s")
    run = pl.kernel(
        _sc_body,
        out_type=jax.ShapeDtypeStruct((_N_ROWS, _N_COLS), x.dtype),
        mesh=mesh,
        scratch_types=[
            pltpu.VMEM((_TPL_ROWS, _N_COLS), jnp.float32),
            pltpu.SemaphoreType.DMA,
        ],
    )
    return run()


_SC_NC = 1
_SC_NW = _SC_NC * _NS
_SC_ROWS_PER_W = _ONES_ROWS // _SC_NW


def _sc_ones_body(o_hbm, tpl, sem):
    c = lax.axis_index("c")
    s = lax.axis_index("s")
    wid = s * _SC_NC + c
    for r in range(_SC_ROWS_PER_W):
        for cc in range(_N_COLS // 16):
            tpl[r, pl.ds(cc * 16, 16)] = jnp.full((16,), 1.0, jnp.float32)
    cp = pltpu.make_async_copy(
        tpl, o_hbm.at[pl.ds(wid * _SC_ROWS_PER_W, _SC_ROWS_PER_W), :], sem
    )
    cp.start()
    cp.wait()


def _sc_ones(dtype):
    mesh = plsc.VectorSubcoreMesh(
        core_axis_name="c", subcore_axis_name="s", num_cores=_SC_NC
    )
    run = pl.kernel(
        _sc_ones_body,
        out_type=jax.ShapeDtypeStruct((_ONES_ROWS, _N_COLS), dtype),
        mesh=mesh,
        scratch_types=[
            pltpu.VMEM((_SC_ROWS_PER_W, _N_COLS), jnp.float32),
            pltpu.SemaphoreType.DMA,
        ],
    )
    return run()


def _tc_zero_init_body(ones_ref, o_ref):
    i = pl.program_id(0)
    o_ref[...] = jnp.zeros(o_ref.shape, jnp.float32)

    @pl.when(i == 0)
    def _():
        o_ref[0:_ONES_ROWS, :] = ones_ref[...]


def _hybrid_kernel(x):
    ones_block = _sc_ones(x.dtype)
    return pl.pallas_call(
        _tc_zero_init_body,
        grid=(_N_ROWS // _BLOCK_ROWS,),
        in_specs=[pl.BlockSpec((_ONES_ROWS, _N_COLS), lambda i: (0, 0))],
        out_specs=pl.BlockSpec((_BLOCK_ROWS, _N_COLS), lambda i: (i, 0)),
        out_shape=jax.ShapeDtypeStruct((_N_ROWS, _N_COLS), x.dtype),
    )(ones_block)


def _tc_zeros_body(o_ref):
    o_ref[...] = jnp.zeros(o_ref.shape, jnp.float32)


def _tc_zeros(dtype):
    return pl.pallas_call(
        _tc_zeros_body,
        grid=(_N_ROWS // _BLOCK_ROWS,),
        out_specs=pl.BlockSpec((_BLOCK_ROWS, _N_COLS), lambda i: (i, 0)),
        out_shape=jax.ShapeDtypeStruct((_N_ROWS, _N_COLS), dtype),
    )()


def _merge_body(z_ref, ones_ref, o_ref):
    del z_ref
    o_ref[...] = ones_ref[...]


def _merge(zeros, ones_block):
    return pl.pallas_call(
        _merge_body,
        grid=(1,),
        in_specs=[
            pl.BlockSpec(memory_space=pl.ANY),
            pl.BlockSpec((_ONES_ROWS, _N_COLS), lambda i: (0, 0)),
        ],
        out_specs=pl.BlockSpec((_ONES_ROWS, _N_COLS), lambda i: (0, 0)),
        out_shape=jax.ShapeDtypeStruct((_N_ROWS, _N_COLS), zeros.dtype),
        input_output_aliases={0: 0},
    )(zeros, ones_block)


def _hybrid2_kernel(x):
    ones_block = _sc_ones(x.dtype)
    zeros = _tc_zeros(x.dtype)
    return _merge(zeros, ones_block)


def kernel(x):
    return _hybrid2_kernel(x)
